# baseline (speedup 1.0000x reference)
import functools

import jax
import jax.numpy as jnp
from jax import lax
from jax.experimental import pallas as pl
from jax.experimental.pallas import tpu as pltpu
from jax.experimental.pallas import tpu_sc as plsc

_BK = 128
_NW = 32
_GW = 16


def _gates_routing_kernel(t_ref, x_ref, Wg1_ref, bg1_ref, Wg2_ref, bg2_ref,
                          gates_ref, g0_ref, g1_ref, pos0_ref, pos1_ref,
                          bexp_ref, bval_ref, bseg_ref, sgex_ref, nsg_ref,
                          mask_scr, rank_scr, cnt_scr):
    i = pl.program_id(0)
    nb_steps = pl.num_programs(0) - 1
    nb = gates_ref.shape[0]
    e_dim = gates_ref.shape[1]
    g_blocks = bexp_ref.shape[1]
    bk_f = jnp.float32(_BK)

    @pl.when(i == 0)
    def _():
        cnt_scr[...] = jnp.zeros_like(cnt_scr)

    @pl.when(i < nb_steps)
    def _():
        xb = x_ref[...].astype(jnp.bfloat16)
        gh = jnp.dot(xb, Wg1_ref[...].astype(jnp.bfloat16),
                     preferred_element_type=jnp.float32)
        gh = jnp.maximum(gh + bg1_ref[...], 0.0)
        logits = jnp.dot(gh.astype(jnp.bfloat16),
                         Wg2_ref[...].astype(jnp.bfloat16),
                         preferred_element_type=jnp.float32)
        logits = (logits + bg2_ref[...]) / t_ref[0]

        iota = lax.broadcasted_iota(jnp.int32, logits.shape, 1)
        m1 = jnp.max(logits, axis=-1, keepdims=True)
        eq1 = logits == m1
        idx1 = jnp.min(jnp.where(eq1, iota, e_dim), axis=-1, keepdims=True)
        first = iota == idx1
        l2 = jnp.where(first, -jnp.inf, logits)
        m2 = jnp.max(l2, axis=-1, keepdims=True)
        eq2 = l2 == m2
        idx2 = jnp.min(jnp.where(eq2, iota, e_dim), axis=-1, keepdims=True)
        second = iota == idx2
        b = jnp.exp(m2 - m1)
        denom = 1.0 + b
        g1 = 1.0 / denom
        g2 = b / denom
        gates_ref[...] = jnp.where(first, g1, 0.0) + jnp.where(second, g2, 0.0)
        g0_ref[...] = jnp.broadcast_to(g1, g0_ref.shape)
        g1_ref[...] = jnp.broadcast_to(g2, g1_ref.shape)

        ind = first.astype(jnp.float32) + second.astype(jnp.float32)
        r_iota = lax.broadcasted_iota(jnp.int32, (nb, nb), 0)
        c_iota = lax.broadcasted_iota(jnp.int32, (nb, nb), 1)
        tril = (r_iota > c_iota).astype(jnp.float32)
        local_excl = jnp.dot(tril, ind, preferred_element_type=jnp.float32)
        row = pl.ds(i * nb, nb)
        mask_scr[row, :] = first.astype(jnp.float32) * 1.0 + \
            second.astype(jnp.float32) * 2.0
        rank_scr[row, :] = cnt_scr[...] + local_excl
        cnt_scr[...] += jnp.sum(ind, axis=0, keepdims=True)

    @pl.when(i == nb_steps)
    def _():
        c = cnt_scr[...]
        padded = jnp.floor((c + (bk_f - 1.0)) * (1.0 / bk_f)) * bk_f
        jj = lax.broadcasted_iota(jnp.int32, (e_dim, e_dim), 0)
        ee = lax.broadcasted_iota(jnp.int32, (e_dim, e_dim), 1)
        mstrict = (jj < ee).astype(jnp.float32)
        off = jnp.dot(padded, mstrict,
                      preferred_element_type=jnp.float32)
        slot = off + rank_scr[...]
        mask = mask_scr[...]
        p0 = jnp.sum(jnp.where(mask == 1.0, slot, 0.0), axis=1,
                     keepdims=True)
        p1 = jnp.sum(jnp.where(mask == 2.0, slot, 0.0), axis=1,
                     keepdims=True)
        pos0_ref[...] = p0.astype(jnp.int32)
        pos1_ref[...] = p1.astype(jnp.int32)

        seg_end = off + padded
        gi = lax.broadcasted_iota(jnp.int32, (1, g_blocks), 1).astype(
            jnp.float32) * bk_f
        lane = lax.broadcasted_iota(jnp.int32, (1, e_dim), 1)
        bexp_f = jnp.zeros((1, g_blocks), jnp.float32)
        for e in range(e_dim):
            end_e = jnp.sum(jnp.where(lane == e, seg_end, 0.0))
            bexp_f = bexp_f + (gi >= end_e).astype(jnp.float32)
        total = jnp.sum(padded)
        bexp_f = jnp.minimum(bexp_f, e_dim - 1)
        bexp_ref[...] = bexp_f.astype(jnp.int32)
        bval_ref[...] = (gi < total).astype(jnp.int32)

        shifted = jnp.concatenate([bexp_f[:, :1], bexp_f[:, :-1]], axis=1)
        chg = (bexp_f != shifted).astype(jnp.float32)
        jjg = lax.broadcasted_iota(jnp.int32, (g_blocks, g_blocks), 0)
        eeg = lax.broadcasted_iota(jnp.int32, (g_blocks, g_blocks), 1)
        mincl = (jjg <= eeg).astype(jnp.float32)
        bseg_ref[...] = jnp.dot(chg, mincl,
                                preferred_element_type=jnp.float32
                                ).astype(jnp.int32)
        nz = (padded > 0.0).astype(jnp.float32)
        rank_e = jnp.dot(nz, mstrict, preferred_element_type=jnp.float32)
        lane_f = lane.astype(jnp.float32)
        sgex_f = jnp.zeros((1, e_dim), jnp.float32)
        for e in range(e_dim):
            r_e = jnp.sum(jnp.where(lane == e, rank_e, 0.0))
            nz_e = jnp.sum(jnp.where(lane == e, nz, 0.0))
            sgex_f = sgex_f + jnp.where(lane_f == r_e, nz_e * e, 0.0)
        sgex_ref[...] = sgex_f.astype(jnp.int32)
        nsg_ref[...] = jnp.reshape(jnp.sum(nz), (1, 1)).astype(jnp.int32)


def _experts_kernel(bexp_ref, bval_ref, bseg_ref, sgex_ref, nsg_ref,
                    xs_ref, W1_ref, b1_ref, W2_ref, b2_ref, ys_ref,
                    w1buf, w2buf, sem1, sem2):
    i = pl.program_id(0)
    seg = bseg_ref[0, i]
    par = lax.rem(seg, 2)
    prev_seg = bseg_ref[0, jnp.maximum(i - 1, 0)]
    seg_start = jnp.logical_or(i == 0, seg != prev_seg)
    nsg = nsg_ref[0, 0]

    @pl.when(i == 0)
    def _():
        e0 = sgex_ref[0, 0]
        pltpu.make_async_copy(W1_ref.at[pl.ds(e0, 1)],
                              w1buf.at[pl.ds(0, 1)], sem1.at[0]).start()
        pltpu.make_async_copy(W2_ref.at[pl.ds(e0, 1)],
                              w2buf.at[pl.ds(0, 1)], sem2.at[0]).start()

    for p in (0, 1):
        @pl.when(jnp.logical_and(seg_start, par == p))
        def _(p=p):
            e_cur = sgex_ref[0, seg]
            pltpu.make_async_copy(W1_ref.at[pl.ds(e_cur, 1)],
                                  w1buf.at[pl.ds(p, 1)], sem1.at[p]).wait()
            pltpu.make_async_copy(W2_ref.at[pl.ds(e_cur, 1)],
                                  w2buf.at[pl.ds(p, 1)], sem2.at[p]).wait()

            @pl.when(seg + 1 < nsg)
            def _():
                e_nx = sgex_ref[0, seg + 1]
                pltpu.make_async_copy(
                    W1_ref.at[pl.ds(e_nx, 1)], w1buf.at[pl.ds(1 - p, 1)],
                    sem1.at[1 - p]).start()
                pltpu.make_async_copy(
                    W2_ref.at[pl.ds(e_nx, 1)], w2buf.at[pl.ds(1 - p, 1)],
                    sem2.at[1 - p]).start()

        @pl.when(jnp.logical_and(bval_ref[0, i] == 1, par == p))
        def _(p=p):
            h1 = jnp.dot(xs_ref[...], w1buf[p],
                         preferred_element_type=jnp.float32)
            h1 = jnp.maximum(h1 + b1_ref[0], 0.0)
            y = jnp.dot(h1, w2buf[p],
                        preferred_element_type=jnp.float32) + b2_ref[0]
            ys_ref[...] = y


def _make_dispatch(n, d, s_slots):
    ch = n // _NW
    mesh = plsc.VectorSubcoreMesh(core_axis_name="c", subcore_axis_name="s",
                                 num_cores=2, num_subcores=16)

    @functools.partial(
        pl.kernel,
        out_type=jax.ShapeDtypeStruct((s_slots, d), jnp.float32),
        mesh=mesh,
        scratch_types=[
            pltpu.VMEM((ch,), jnp.int32),
            pltpu.VMEM((ch,), jnp.int32),
            pltpu.VMEM((ch, d), jnp.float32),
            pltpu.SemaphoreType.DMA,
        ],
    )
    def dispatch(x_hbm, i0_hbm, i1_hbm, xs_hbm, i0_v, i1_v, rows_v, sem):
        wid = lax.axis_index("s") * 2 + lax.axis_index("c")
        base = wid * ch
        sl = pl.ds(base, ch)
        c1 = pltpu.async_copy(x_hbm.at[sl], rows_v, sem)
        c2 = pltpu.async_copy(i0_hbm.at[sl], i0_v, sem)
        c3 = pltpu.async_copy(i1_hbm.at[sl], i1_v, sem)
        c1.wait(); c2.wait(); c3.wait()
        s1 = pltpu.async_copy(rows_v, xs_hbm.at[i0_v], sem)
        s2 = pltpu.async_copy(rows_v, xs_hbm.at[i1_v], sem)
        s1.wait(); s2.wait()

    return dispatch


def _make_combine(n, h_dim):
    ch = n // _NW
    win = 32
    n_win = ch // win
    mesh = plsc.VectorSubcoreMesh(core_axis_name="c", subcore_axis_name="s",
                                 num_cores=2, num_subcores=16)

    @functools.partial(
        pl.kernel,
        out_type=jax.ShapeDtypeStruct((n, h_dim), jnp.float32),
        mesh=mesh,
        scratch_types=[
            pltpu.VMEM((ch,), jnp.int32),
            pltpu.VMEM((ch,), jnp.int32),
            pltpu.VMEM((ch, 16), jnp.float32),
            pltpu.VMEM((ch, 16), jnp.float32),
            pltpu.VMEM((win, h_dim), jnp.float32),
            pltpu.VMEM((win, h_dim), jnp.float32),
            pltpu.VMEM((win, h_dim), jnp.float32),
            pltpu.SemaphoreType.DMA,
        ],
    )
    def combine(ys_hbm, i0_hbm, i1_hbm, g0_hbm, g1_hbm, out_hbm,
                i0_v, i1_v, ga_v, gb_v, a_v, b_v, o_v, sem):
        wid = lax.axis_index("s") * 2 + lax.axis_index("c")
        base = wid * ch
        c1 = pltpu.async_copy(i0_hbm.at[pl.ds(base, ch)], i0_v, sem)
        c2 = pltpu.async_copy(i1_hbm.at[pl.ds(base, ch)], i1_v, sem)
        c3 = pltpu.async_copy(g0_hbm.at[pl.ds(base, ch)], ga_v, sem)
        c4 = pltpu.async_copy(g1_hbm.at[pl.ds(base, ch)], gb_v, sem)
        c1.wait(); c2.wait(); c3.wait(); c4.wait()

        @pl.loop(0, n_win)
        def _(w):
            wbase = base + w * win
            ca = pltpu.async_copy(
                ys_hbm.at[i0_v.at[pl.ds(w * win, win)]], a_v, sem)
            cb = pltpu.async_copy(
                ys_hbm.at[i1_v.at[pl.ds(w * win, win)]], b_v, sem)
            ca.wait(); cb.wait()

            @pl.loop(0, win)
            def _(r):
                ga = ga_v[w * win + r]
                gb = gb_v[w * win + r]

                @pl.loop(0, h_dim // 16, unroll=8)
                def _(cc):
                    sl = pl.ds(cc * 16, 16)
                    o_v[r, sl] = a_v[r, sl] * ga + b_v[r, sl] * gb

            pltpu.sync_copy(o_v, out_hbm.at[pl.ds(wbase, win)])

    return combine


def kernel(x, W1, b1, W2, b2, Wg1, bg1, Wg2, bg2, temperature):
    n, d = x.shape
    e_num, _, h_dim = W1.shape
    k_top = 2
    g_blocks = (n * k_top) // _BK + e_num
    s_slots = g_blocks * _BK
    nb_steps = 4
    nb = n // nb_steps

    t = jnp.reshape(temperature.astype(jnp.float32), (1,))
    bg1_2d = jnp.reshape(bg1, (1, h_dim))
    bg2_2d = jnp.reshape(bg2, (1, e_num))
    b1_3d = jnp.reshape(b1, (e_num, 1, h_dim))
    b2_3d = jnp.reshape(b2, (e_num, 1, h_dim))

    (gates, g0, g1, pos0, pos1, bexp, bval, bseg, sgex,
     nsg) = pl.pallas_call(
        _gates_routing_kernel,
        grid=(nb_steps + 1,),
        in_specs=[
            pl.BlockSpec(memory_space=pltpu.SMEM),
            pl.BlockSpec((nb, d), lambda i: (jnp.minimum(i, nb_steps - 1), 0)),
            pl.BlockSpec((d, h_dim), lambda i: (0, 0)),
            pl.BlockSpec((1, h_dim), lambda i: (0, 0)),
            pl.BlockSpec((h_dim, e_num), lambda i: (0, 0)),
            pl.BlockSpec((1, e_num), lambda i: (0, 0)),
        ],
        out_specs=[
            pl.BlockSpec((nb, e_num),
                         lambda i: (jnp.minimum(i, nb_steps - 1), 0)),
            pl.BlockSpec((nb, 16),
                         lambda i: (jnp.minimum(i, nb_steps - 1), 0)),
            pl.BlockSpec((nb, 16),
                         lambda i: (jnp.minimum(i, nb_steps - 1), 0)),
            pl.BlockSpec((n, 1), lambda i: (0, 0)),
            pl.BlockSpec((n, 1), lambda i: (0, 0)),
            pl.BlockSpec((1, g_blocks), lambda i: (0, 0)),
            pl.BlockSpec((1, g_blocks), lambda i: (0, 0)),
            pl.BlockSpec((1, g_blocks), lambda i: (0, 0)),
            pl.BlockSpec((1, e_num), lambda i: (0, 0)),
            pl.BlockSpec((1, 1), lambda i: (0, 0)),
        ],
        out_shape=[
            jax.ShapeDtypeStruct((n, e_num), jnp.float32),
            jax.ShapeDtypeStruct((n, 16), jnp.float32),
            jax.ShapeDtypeStruct((n, 16), jnp.float32),
            jax.ShapeDtypeStruct((n, 1), jnp.int32),
            jax.ShapeDtypeStruct((n, 1), jnp.int32),
            jax.ShapeDtypeStruct((1, g_blocks), jnp.int32),
            jax.ShapeDtypeStruct((1, g_blocks), jnp.int32),
            jax.ShapeDtypeStruct((1, g_blocks), jnp.int32),
            jax.ShapeDtypeStruct((1, e_num), jnp.int32),
            jax.ShapeDtypeStruct((1, 1), jnp.int32),
        ],
        scratch_shapes=[
            pltpu.VMEM((n, e_num), jnp.float32),
            pltpu.VMEM((n, e_num), jnp.float32),
            pltpu.VMEM((1, e_num), jnp.float32),
        ],
        compiler_params=pltpu.CompilerParams(
            dimension_semantics=("arbitrary",),
        ),
    )(t, x, Wg1, bg1_2d, Wg2, bg2_2d)

    idx0 = jnp.reshape(pos0, (n,))
    idx1 = jnp.reshape(pos1, (n,))

    xs = _make_dispatch(n, d, s_slots)(x, idx0, idx1)

    grid_spec = pltpu.PrefetchScalarGridSpec(
        num_scalar_prefetch=5,
        grid=(g_blocks,),
        in_specs=[
            pl.BlockSpec((_BK, d), lambda i, *_: (i, 0)),
            pl.BlockSpec(memory_space=pl.ANY),
            pl.BlockSpec((1, 1, h_dim), lambda i, be, *_: (be[0, i], 0, 0)),
            pl.BlockSpec(memory_space=pl.ANY),
            pl.BlockSpec((1, 1, h_dim), lambda i, be, *_: (be[0, i], 0, 0)),
        ],
        out_specs=pl.BlockSpec((_BK, h_dim), lambda i, *_: (i, 0)),
        scratch_shapes=[
            pltpu.VMEM((2, d, h_dim), jnp.float32),
            pltpu.VMEM((2, h_dim, h_dim), jnp.float32),
            pltpu.SemaphoreType.DMA((2,)),
            pltpu.SemaphoreType.DMA((2,)),
        ],
    )
    ys = pl.pallas_call(
        _experts_kernel,
        grid_spec=grid_spec,
        out_shape=jax.ShapeDtypeStruct((s_slots, h_dim), jnp.float32),
        compiler_params=pltpu.CompilerParams(
            dimension_semantics=("arbitrary",),
        ),
    )(bexp, bval, bseg, sgex, nsg, xs, W1, b1_3d, W2, b2_3d)

    out = _make_combine(n, h_dim)(ys, idx0, idx1, g0, g1)

    return out, gates


# speedup vs baseline: 1.2937x; 1.2937x over previous
import jax
import jax.numpy as jnp
from jax.experimental import pallas as pl
from jax.experimental.pallas import tpu as pltpu


def _gates_kernel(t_ref, x_ref, Wg1_ref, bg1_ref, Wg2_ref, bg2_ref, gates_ref):
    x = x_ref[...].astype(jnp.bfloat16)
    gh = jnp.dot(x, Wg1_ref[...].astype(jnp.bfloat16),
                 preferred_element_type=jnp.float32)
    gh = jnp.maximum(gh + bg1_ref[...], 0.0)
    logits = jnp.dot(gh.astype(jnp.bfloat16),
                     Wg2_ref[...].astype(jnp.bfloat16),
                     preferred_element_type=jnp.float32)
    logits = (logits + bg2_ref[...]) / t_ref[0]

    e_dim = logits.shape[-1]
    iota = jax.lax.broadcasted_iota(jnp.int32, logits.shape, 1)
    m1 = jnp.max(logits, axis=-1, keepdims=True)
    eq1 = logits == m1
    idx1 = jnp.min(jnp.where(eq1, iota, e_dim), axis=-1, keepdims=True)
    first = iota == idx1
    l2 = jnp.where(first, -jnp.inf, logits)
    m2 = jnp.max(l2, axis=-1, keepdims=True)
    eq2 = l2 == m2
    idx2 = jnp.min(jnp.where(eq2, iota, e_dim), axis=-1, keepdims=True)
    second = iota == idx2
    b = jnp.exp(m2 - m1)
    denom = 1.0 + b
    g1 = 1.0 / denom
    g2 = b / denom
    gates_ref[...] = jnp.where(first, g1, 0.0) + jnp.where(second, g2, 0.0)


def _experts_kernel(x_ref, W1_ref, b1_ref, W2_ref, b2_ref, gates_ref, out_ref):
    e = pl.program_id(0)
    j = pl.program_id(1)
    eiota = jax.lax.broadcasted_iota(jnp.int32, gates_ref.shape, 1)
    g = jnp.sum(jnp.where(eiota == e, gates_ref[...], 0.0), axis=1,
                keepdims=True)
    h = jnp.dot(x_ref[...], W1_ref[0],
                preferred_element_type=jnp.float32)
    h = jnp.maximum(h + b1_ref[0], 0.0)
    contrib = g * jnp.dot(h, W2_ref[0],
                          preferred_element_type=jnp.float32)

    @pl.when(jnp.logical_and(e == 0, j == 0))
    def _():
        out_ref[...] = contrib + g * b2_ref[0]

    @pl.when(jnp.logical_and(e > 0, j == 0))
    def _():
        out_ref[...] += contrib + g * b2_ref[0]

    @pl.when(j > 0)
    def _():
        out_ref[...] += contrib


def kernel(x, W1, b1, W2, b2, Wg1, bg1, Wg2, bg2, temperature):
    n, d = x.shape
    e_num, _, h_dim = W1.shape
    t = jnp.reshape(temperature.astype(jnp.float32), (1,))
    bg1_2d = jnp.reshape(bg1, (1, h_dim))
    bg2_2d = jnp.reshape(bg2, (1, e_num))
    b1_3d = jnp.reshape(b1, (e_num, 1, h_dim))
    b2_3d = jnp.reshape(b2, (e_num, 1, h_dim))

    nb_gate = 512
    gates = pl.pallas_call(
        _gates_kernel,
        grid=(n // nb_gate,),
        in_specs=[
            pl.BlockSpec(memory_space=pltpu.SMEM),
            pl.BlockSpec((nb_gate, d), lambda i: (i, 0)),
            pl.BlockSpec((d, h_dim), lambda i: (0, 0)),
            pl.BlockSpec((1, h_dim), lambda i: (0, 0)),
            pl.BlockSpec((h_dim, e_num), lambda i: (0, 0)),
            pl.BlockSpec((1, e_num), lambda i: (0, 0)),
        ],
        out_specs=pl.BlockSpec((nb_gate, e_num), lambda i: (i, 0)),
        out_shape=jax.ShapeDtypeStruct((n, e_num), jnp.float32),
        compiler_params=pltpu.CompilerParams(
            dimension_semantics=("arbitrary",),
        ),
    )(t, x, Wg1, bg1_2d, Wg2, bg2_2d)

    hc = 512
    b1_chunks = jnp.reshape(b1, (e_num, 1, h_dim))
    out = pl.pallas_call(
        _experts_kernel,
        grid=(e_num, h_dim // hc),
        in_specs=[
            pl.BlockSpec((n, d), lambda e, j: (0, 0)),
            pl.BlockSpec((1, d, hc), lambda e, j: (e, 0, j)),
            pl.BlockSpec((1, 1, hc), lambda e, j: (e, 0, j)),
            pl.BlockSpec((1, hc, h_dim), lambda e, j: (e, j, 0)),
            pl.BlockSpec((1, 1, h_dim), lambda e, j: (e, 0, 0)),
            pl.BlockSpec((n, e_num), lambda e, j: (0, 0)),
        ],
        out_specs=pl.BlockSpec((n, h_dim), lambda e, j: (0, 0)),
        out_shape=jax.ShapeDtypeStruct((n, h_dim), jnp.float32),
        compiler_params=pltpu.CompilerParams(
            dimension_semantics=("arbitrary", "arbitrary"),
        ),
    )(x, W1, b1_chunks, W2, b2_3d, gates)

    return out, gates


# speedup vs baseline: 1.3802x; 1.0669x over previous
import jax
import jax.numpy as jnp
from jax.experimental import pallas as pl
from jax.experimental.pallas import tpu as pltpu


def _gates_kernel(t_ref, x_ref, Wg1_ref, bg1_ref, Wg2_ref, bg2_ref, gates_ref):
    x = x_ref[...].astype(jnp.bfloat16)
    gh = jnp.dot(x, Wg1_ref[...].astype(jnp.bfloat16),
                 preferred_element_type=jnp.float32)
    gh = jnp.maximum(gh + bg1_ref[...], 0.0)
    logits = jnp.dot(gh.astype(jnp.bfloat16),
                     Wg2_ref[...].astype(jnp.bfloat16),
                     preferred_element_type=jnp.float32)
    logits = (logits + bg2_ref[...]) / t_ref[0]

    e_dim = logits.shape[-1]
    iota = jax.lax.broadcasted_iota(jnp.int32, logits.shape, 1)
    m1 = jnp.max(logits, axis=-1, keepdims=True)
    eq1 = logits == m1
    idx1 = jnp.min(jnp.where(eq1, iota, e_dim), axis=-1, keepdims=True)
    first = iota == idx1
    l2 = jnp.where(first, -jnp.inf, logits)
    m2 = jnp.max(l2, axis=-1, keepdims=True)
    eq2 = l2 == m2
    idx2 = jnp.min(jnp.where(eq2, iota, e_dim), axis=-1, keepdims=True)
    second = iota == idx2
    b = jnp.exp(m2 - m1)
    denom = 1.0 + b
    g1 = 1.0 / denom
    g2 = b / denom
    gates_ref[...] = jnp.where(first, g1, 0.0) + jnp.where(second, g2, 0.0)


def _experts_kernel(x_ref, W1_ref, b1_ref, W2_ref, b2_ref, gates_ref, out_ref):
    e = pl.program_id(0)
    x = x_ref[...]
    h = jnp.dot(x, W1_ref[0], precision=jax.lax.Precision.DEFAULT,
                preferred_element_type=jnp.float32)
    h = jnp.maximum(h + b1_ref[0], 0.0)
    y = jnp.dot(h, W2_ref[0], precision=jax.lax.Precision.DEFAULT,
                preferred_element_type=jnp.float32)
    y = y + b2_ref[0]
    eiota = jax.lax.broadcasted_iota(jnp.int32, gates_ref.shape, 1)
    g = jnp.sum(jnp.where(eiota == e, gates_ref[...], 0.0), axis=1,
                keepdims=True)
    contrib = g * y

    @pl.when(e == 0)
    def _():
        out_ref[...] = contrib

    @pl.when(e > 0)
    def _():
        out_ref[...] += contrib


def kernel(x, W1, b1, W2, b2, Wg1, bg1, Wg2, bg2, temperature):
    n, d = x.shape
    e_num, _, h_dim = W1.shape
    t = jnp.reshape(temperature.astype(jnp.float32), (1,))
    bg1_2d = jnp.reshape(bg1, (1, h_dim))
    bg2_2d = jnp.reshape(bg2, (1, e_num))
    b1_3d = jnp.reshape(b1, (e_num, 1, h_dim))
    b2_3d = jnp.reshape(b2, (e_num, 1, h_dim))

    nb_gate = 512
    gates = pl.pallas_call(
        _gates_kernel,
        grid=(n // nb_gate,),
        in_specs=[
            pl.BlockSpec(memory_space=pltpu.SMEM),
            pl.BlockSpec((nb_gate, d), lambda i: (i, 0)),
            pl.BlockSpec((d, h_dim), lambda i: (0, 0)),
            pl.BlockSpec((1, h_dim), lambda i: (0, 0)),
            pl.BlockSpec((h_dim, e_num), lambda i: (0, 0)),
            pl.BlockSpec((1, e_num), lambda i: (0, 0)),
        ],
        out_specs=pl.BlockSpec((nb_gate, e_num), lambda i: (i, 0)),
        out_shape=jax.ShapeDtypeStruct((n, e_num), jnp.float32),
        compiler_params=pltpu.CompilerParams(
            dimension_semantics=("arbitrary",),
        ),
    )(t, x, Wg1, bg1_2d, Wg2, bg2_2d)

    out = pl.pallas_call(
        _experts_kernel,
        grid=(e_num,),
        in_specs=[
            pl.BlockSpec((n, d), lambda e: (0, 0)),
            pl.BlockSpec((1, d, h_dim), lambda e: (e, 0, 0)),
            pl.BlockSpec((1, 1, h_dim), lambda e: (e, 0, 0)),
            pl.BlockSpec((1, h_dim, h_dim), lambda e: (e, 0, 0)),
            pl.BlockSpec((1, 1, h_dim), lambda e: (e, 0, 0)),
            pl.BlockSpec((n, e_num), lambda e: (0, 0)),
        ],
        out_specs=pl.BlockSpec((n, h_dim), lambda e: (0, 0)),
        out_shape=jax.ShapeDtypeStruct((n, h_dim), jnp.float32),
        compiler_params=pltpu.CompilerParams(
            dimension_semantics=("arbitrary",),
        ),
    )(x, W1, b1_3d, W2, b2_3d, gates)

    return out, gates
